# Initial kernel scaffold; baseline (speedup 1.0000x reference)
import jax
import jax.numpy as jnp
from jax.experimental import pallas as pl


def kernel(x, edge_index, W_mu, b_mu, W_logstd, b_logstd):
    raise NotImplementedError("write your pallas kernel here")


# speedup vs baseline: 31.4799x; 31.4799x over previous
import functools

import jax
import jax.numpy as jnp
from jax import lax
from jax.experimental import pallas as pl
from jax.experimental.pallas import tpu as pltpu
from jax.experimental.pallas import tpu_sc as plsc

N = 10000
E = 320000
D = 128
D_OUT = 64

NC = 2
NS = 16
NW = NC * NS
EPW = E // NW
CH = 80
NCHUNK = EPW // CH
NP = 10240
RPS = NP // NS

@functools.cache
def _make_sc_degree():
    mesh = plsc.VectorSubcoreMesh(core_axis_name="c", subcore_axis_name="s")

    @functools.partial(
        pl.kernel,
        out_type=jax.ShapeDtypeStruct((NC * NP, 16), jnp.float32),
        mesh=mesh,
        scratch_types=[
            pltpu.VMEM((NCHUNK, CH), jnp.int32),
            pltpu.VMEM((CH, 16), jnp.float32),
            pltpu.VMEM_SHARED((NP, 16), jnp.float32),
        ],
    )
    def _sc_degree(dst3, ones_hbm, zeros_hbm, out_hbm, didx, ones_v, acc):
        cid = lax.axis_index("c")
        sid = lax.axis_index("s")
        wid = cid * NS + sid
        r0 = pl.multiple_of(sid * RPS, 8)
        pltpu.sync_copy(zeros_hbm.at[pl.ds(r0, RPS)], acc.at[pl.ds(r0, RPS)])
        pltpu.sync_copy(ones_hbm, ones_v)
        pltpu.sync_copy(dst3.at[wid], didx)
        plsc.subcore_barrier()

        def body(j, carry):
            pltpu.sync_copy(ones_v, acc.at[didx.at[j]], add=True)
            return carry

        lax.fori_loop(0, NCHUNK, body, 0)
        plsc.subcore_barrier()
        pltpu.sync_copy(acc.at[pl.ds(r0, RPS)], out_hbm.at[pl.ds(cid * NP + r0, RPS)])

    return _sc_degree


@functools.cache
def _make_sc_agg():
    mesh = plsc.VectorSubcoreMesh(core_axis_name="c", subcore_axis_name="s")

    @functools.partial(
        pl.kernel,
        out_type=jax.ShapeDtypeStruct((NC * NP, D), jnp.float32),
        mesh=mesh,
        scratch_types=[
            pltpu.VMEM((NCHUNK, CH), jnp.int32),
            pltpu.VMEM((NCHUNK, CH), jnp.int32),
            pltpu.VMEM((CH, D), jnp.float32),
            pltpu.VMEM_SHARED((NP, D), jnp.float32),
            pltpu.SemaphoreType.DMA,
        ],
    )
    def _sc_agg(src3, dst3, z_hbm, zeros_hbm, out_hbm, sidx, didx, rows, acc, sem):
        cid = lax.axis_index("c")
        sid = lax.axis_index("s")
        wid = cid * NS + sid
        r0 = pl.multiple_of(sid * RPS, 8)
        pltpu.sync_copy(zeros_hbm.at[pl.ds(r0, RPS)], acc.at[pl.ds(r0, RPS)])
        pltpu.sync_copy(src3.at[wid], sidx)
        pltpu.sync_copy(dst3.at[wid], didx)
        plsc.subcore_barrier()

        def body(j, carry):
            pltpu.async_copy(z_hbm.at[sidx.at[j]], rows, sem).wait()
            pltpu.sync_copy(rows, acc.at[didx.at[j]], add=True)
            return carry

        lax.fori_loop(0, NCHUNK, body, 0)
        plsc.subcore_barrier()
        pltpu.sync_copy(acc.at[pl.ds(r0, RPS)], out_hbm.at[pl.ds(cid * NP + r0, RPS)])

    return _sc_agg


def _tc_prescale_body(degp_ref, x_ref, z_ref):
    a = degp_ref[...]
    deg = a[0:N, 0:1] + a[NP : NP + N, 0:1] + 1.0
    dinv = lax.rsqrt(deg)
    z_ref[...] = x_ref[...] * dinv


def _tc_final_body(aggp_ref, x_ref, degp_ref, w_ref, b_ref, out_ref):
    a = degp_ref[...]
    deg = a[0:N, 0:1] + a[NP : NP + N, 0:1] + 1.0
    dinv = lax.rsqrt(deg)
    g = aggp_ref[0:N, :] + aggp_ref[NP : NP + N, :]
    v = g * dinv + x_ref[...] * (dinv * dinv)
    out_ref[...] = (
        jnp.dot(v, w_ref[...], preferred_element_type=jnp.float32) + b_ref[...]
    )


def kernel(x, edge_index, W_mu, b_mu, W_logstd, b_logstd):
    src3 = edge_index[0].reshape(NW, NCHUNK, CH)
    dst3 = edge_index[1].reshape(NW, NCHUNK, CH)
    ones16 = jnp.ones((CH, 16), jnp.float32)
    zeros16 = jnp.zeros((NP, 16), jnp.float32)
    zerosD = jnp.zeros((NP, D), jnp.float32)

    degp = _make_sc_degree()(dst3, ones16, zeros16)
    z = pl.pallas_call(
        _tc_prescale_body,
        out_shape=jax.ShapeDtypeStruct((N, D), jnp.float32),
    )(degp, x)
    aggp = _make_sc_agg()(src3, dst3, z, zerosD)

    W_cat = jnp.concatenate([W_mu, W_logstd], axis=1)
    b_cat = jnp.concatenate([b_mu, b_logstd]).reshape(1, 2 * D_OUT)
    out = pl.pallas_call(
        _tc_final_body,
        out_shape=jax.ShapeDtypeStruct((N, 2 * D_OUT), jnp.float32),
    )(aggp, x, degp, W_cat, b_cat)
    return out[:, :D_OUT], out[:, D_OUT:]


# speedup vs baseline: 41.7577x; 1.3265x over previous
import functools

import jax
import jax.numpy as jnp
from jax import lax
from jax.experimental import pallas as pl
from jax.experimental.pallas import tpu as pltpu
from jax.experimental.pallas import tpu_sc as plsc

N = 10000
E = 320000
D = 128
D_OUT = 64

NC = 2
NS = 16
NW = NC * NS
CH = 128
NCHUNK = 80
EPW = NCHUNK * CH
EP = NW * EPW
NP = 10240
RPS = NP // NS


@functools.cache
def _make_sc_degree():
    mesh = plsc.VectorSubcoreMesh(core_axis_name="c", subcore_axis_name="s")

    @functools.partial(
        pl.kernel,
        out_type=jax.ShapeDtypeStruct((NC * NP,), jnp.float32),
        mesh=mesh,
        scratch_types=[
            pltpu.VMEM((CH,), jnp.int32),
            pltpu.VMEM((CH,), jnp.int32),
            pltpu.VMEM((CH,), jnp.float32),
            pltpu.VMEM_SHARED((NP,), jnp.float32),
            pltpu.SemaphoreType.DMA,
            pltpu.SemaphoreType.DMA,
        ],
    )
    def _sc_degree(dst_hbm, ones_hbm, zeros_hbm, out_hbm,
                   didx0, didx1, ones_v, acc, isem0, isem1):
        cid = lax.axis_index("c")
        sid = lax.axis_index("s")
        wid = cid * NS + sid
        ebase = wid * EPW
        r0 = pl.multiple_of(sid * RPS, 8)
        pltpu.sync_copy(zeros_hbm.at[pl.ds(r0, RPS)], acc.at[pl.ds(r0, RPS)])
        pltpu.sync_copy(ones_hbm, ones_v)

        def idx_start(j, dv, isem):
            base = pl.multiple_of(ebase + j * CH, 8)
            pltpu.async_copy(dst_hbm.at[pl.ds(base, CH)], dv, isem)

        def idx_wait(dv, isem):
            pltpu.make_async_copy(dst_hbm.at[pl.ds(0, CH)], dv, isem).wait()

        plsc.subcore_barrier()
        idx_start(0, didx0, isem0)

        def body(i, carry):
            j0 = 2 * i
            j1 = j0 + 1
            idx_start(j1, didx1, isem1)
            idx_wait(didx0, isem0)
            pltpu.sync_copy(ones_v, acc.at[didx0], add=True)

            @pl.when(i < NCHUNK // 2 - 1)
            def _():
                idx_start(j0 + 2, didx0, isem0)

            idx_wait(didx1, isem1)
            pltpu.sync_copy(ones_v, acc.at[didx1], add=True)
            return carry

        lax.fori_loop(0, NCHUNK // 2, body, 0)
        plsc.subcore_barrier()
        pltpu.sync_copy(acc.at[pl.ds(r0, RPS)], out_hbm.at[pl.ds(cid * NP + r0, RPS)])

    return _sc_degree


@functools.cache
def _make_sc_agg():
    mesh = plsc.VectorSubcoreMesh(core_axis_name="c", subcore_axis_name="s")

    @functools.partial(
        pl.kernel,
        out_type=jax.ShapeDtypeStruct((NC * NP, D), jnp.float32),
        mesh=mesh,
        scratch_types=[
            pltpu.VMEM((CH,), jnp.int32),
            pltpu.VMEM((CH,), jnp.int32),
            pltpu.VMEM((CH,), jnp.int32),
            pltpu.VMEM((CH,), jnp.int32),
            pltpu.VMEM((CH, D), jnp.float32),
            pltpu.VMEM((CH, D), jnp.float32),
            pltpu.VMEM_SHARED((NP, D), jnp.float32),
            pltpu.SemaphoreType.DMA,
            pltpu.SemaphoreType.DMA,
            pltpu.SemaphoreType.DMA,
            pltpu.SemaphoreType.DMA,
        ],
    )
    def _sc_agg(src_hbm, dst_hbm, z_hbm, zeros_hbm, out_hbm,
                sidx0, sidx1, didx0, didx1, rows0, rows1, acc,
                isem0, isem1, gsem0, gsem1):
        cid = lax.axis_index("c")
        sid = lax.axis_index("s")
        wid = cid * NS + sid
        ebase = wid * EPW
        r0 = pl.multiple_of(sid * RPS, 8)
        pltpu.sync_copy(zeros_hbm.at[pl.ds(r0, RPS)], acc.at[pl.ds(r0, RPS)])

        def idx_start(j, sv, dv, isem):
            base = pl.multiple_of(ebase + j * CH, 8)
            pltpu.async_copy(src_hbm.at[pl.ds(base, CH)], sv, isem)
            pltpu.async_copy(dst_hbm.at[pl.ds(base, CH)], dv, isem)

        def idx_wait(sv, dv, isem):
            pltpu.make_async_copy(src_hbm.at[pl.ds(0, CH)], sv, isem).wait()
            pltpu.make_async_copy(dst_hbm.at[pl.ds(0, CH)], dv, isem).wait()

        def gather_start(sv, rv, gsem):
            pltpu.async_copy(z_hbm.at[sv], rv, gsem)

        def gather_wait(sv, rv, gsem):
            pltpu.make_async_copy(z_hbm.at[sv], rv, gsem).wait()

        plsc.subcore_barrier()

        idx_start(0, sidx0, didx0, isem0)
        idx_start(1, sidx1, didx1, isem1)
        idx_wait(sidx0, didx0, isem0)
        gather_start(sidx0, rows0, gsem0)

        def body(i, carry):
            j0 = 2 * i
            j1 = j0 + 1
            idx_wait(sidx1, didx1, isem1)
            gather_start(sidx1, rows1, gsem1)
            gather_wait(sidx0, rows0, gsem0)
            pltpu.sync_copy(rows0, acc.at[didx0], add=True)

            @pl.when(i < NCHUNK // 2 - 1)
            def _():
                idx_start(j0 + 2, sidx0, didx0, isem0)
                idx_wait(sidx0, didx0, isem0)
                gather_start(sidx0, rows0, gsem0)

            gather_wait(sidx1, rows1, gsem1)
            pltpu.sync_copy(rows1, acc.at[didx1], add=True)

            @pl.when(i < NCHUNK // 2 - 1)
            def _():
                idx_start(j1 + 2, sidx1, didx1, isem1)

            return carry

        lax.fori_loop(0, NCHUNK // 2, body, 0)
        plsc.subcore_barrier()
        pltpu.sync_copy(acc.at[pl.ds(r0, RPS)], out_hbm.at[pl.ds(cid * NP + r0, RPS)])

    return _sc_agg


def _tc_prescale_body(c0_ref, c1_ref, x_ref, z_ref):
    deg = c0_ref[...] + c1_ref[...] + 1.0
    dinv = lax.rsqrt(deg)
    z_ref[0:N, :] = x_ref[...] * dinv
    z_ref[N:NP, :] = jnp.zeros((NP - N, D), jnp.float32)


def _tc_final_body(aggp_ref, x_ref, c0_ref, c1_ref, w_ref, b_ref, out_ref):
    deg = c0_ref[...] + c1_ref[...] + 1.0
    dinv = lax.rsqrt(deg)
    g = aggp_ref[0:N, :] + aggp_ref[NP : NP + N, :]
    v = g * dinv + x_ref[...] * (dinv * dinv)
    out_ref[...] = (
        jnp.dot(v, w_ref[...], preferred_element_type=jnp.float32) + b_ref[...]
    )


def kernel(x, edge_index, W_mu, b_mu, W_logstd, b_logstd):
    pad = N + (jnp.arange(EP - E, dtype=jnp.int32) % (NP - N))
    srcp = jnp.concatenate([edge_index[0], pad])
    dstp = jnp.concatenate([edge_index[1], pad])
    ones1 = jnp.ones((CH,), jnp.float32)
    zeros1 = jnp.zeros((NP,), jnp.float32)
    zerosD = jnp.zeros((NP, D), jnp.float32)

    degp = _make_sc_degree()(dstp, ones1, zeros1)
    c0 = degp[0:N].reshape(N, 1)
    c1 = degp[NP : NP + N].reshape(N, 1)
    z = pl.pallas_call(
        _tc_prescale_body,
        out_shape=jax.ShapeDtypeStruct((NP, D), jnp.float32),
    )(c0, c1, x)
    aggp = _make_sc_agg()(srcp, dstp, z, zerosD)

    W_cat = jnp.concatenate([W_mu, W_logstd], axis=1)
    b_cat = jnp.concatenate([b_mu, b_logstd]).reshape(1, 2 * D_OUT)
    out = pl.pallas_call(
        _tc_final_body,
        out_shape=jax.ShapeDtypeStruct((N, 2 * D_OUT), jnp.float32),
    )(aggp, x, c0, c1, W_cat, b_cat)
    return out[:, :D_OUT], out[:, D_OUT:]


# speedup vs baseline: 49.0141x; 1.1738x over previous
import functools

import jax
import jax.numpy as jnp
from jax import lax
from jax.experimental import pallas as pl
from jax.experimental.pallas import tpu as pltpu
from jax.experimental.pallas import tpu_sc as plsc

N = 10000
E = 320000
D = 128
D_OUT = 64

NC = 2
NS = 16
NW = NC * NS
CH = 128
NCHUNK = 80
EPW = NCHUNK * CH
EP = NW * EPW
NP = 10240
RPS = NP // NS


@functools.cache
def _make_sc_degree():
    mesh = plsc.VectorSubcoreMesh(core_axis_name="c", subcore_axis_name="s")

    @functools.partial(
        pl.kernel,
        out_type=jax.ShapeDtypeStruct((NC * NP,), jnp.float32),
        mesh=mesh,
        scratch_types=[
            pltpu.VMEM((NCHUNK, CH), jnp.int32),
            pltpu.VMEM((CH,), jnp.float32),
            pltpu.VMEM_SHARED((NP,), jnp.float32),
            pltpu.SemaphoreType.DMA,
        ],
    )
    def _sc_degree(dst3, ones_hbm, zeros_hbm, out_hbm, didx, ones_v, acc, sem):
        cid = lax.axis_index("c")
        sid = lax.axis_index("s")
        wid = cid * NS + sid
        r0 = pl.multiple_of(sid * RPS, 8)
        pltpu.sync_copy(zeros_hbm.at[pl.ds(r0, RPS)], acc.at[pl.ds(r0, RPS)])
        pltpu.sync_copy(ones_hbm, ones_v)
        pltpu.sync_copy(dst3.at[wid], didx)
        plsc.subcore_barrier()

        def body(j, carry):
            pltpu.async_copy(ones_v, acc.at[didx.at[j]], sem, add=True)

            @pl.when(j >= 8)
            def _():
                pltpu.make_async_copy(ones_v, acc.at[didx.at[0]], sem).wait()

            return carry

        lax.fori_loop(0, NCHUNK, body, 0)

        def drain(j, carry):
            pltpu.make_async_copy(ones_v, acc.at[didx.at[0]], sem).wait()
            return carry

        lax.fori_loop(0, 8, drain, 0)
        plsc.subcore_barrier()
        pltpu.sync_copy(acc.at[pl.ds(r0, RPS)], out_hbm.at[pl.ds(cid * NP + r0, RPS)])

    return _sc_degree


@functools.cache
def _make_sc_agg():
    mesh = plsc.VectorSubcoreMesh(core_axis_name="c", subcore_axis_name="s")

    @functools.partial(
        pl.kernel,
        out_type=jax.ShapeDtypeStruct((NC * NP, D), jnp.float32),
        mesh=mesh,
        scratch_types=[
            pltpu.VMEM((CH,), jnp.int32),
            pltpu.VMEM((CH,), jnp.int32),
            pltpu.VMEM((CH,), jnp.int32),
            pltpu.VMEM((CH,), jnp.int32),
            pltpu.VMEM((CH, D), jnp.float32),
            pltpu.VMEM((CH, D), jnp.float32),
            pltpu.VMEM_SHARED((NP, D), jnp.float32),
            pltpu.SemaphoreType.DMA,
            pltpu.SemaphoreType.DMA,
            pltpu.SemaphoreType.DMA,
            pltpu.SemaphoreType.DMA,
            pltpu.SemaphoreType.DMA,
            pltpu.SemaphoreType.DMA,
        ],
    )
    def _sc_agg(src_hbm, dst_hbm, z_hbm, zeros_hbm, out_hbm,
                sidx0, sidx1, didx0, didx1, rows0, rows1, acc,
                ssem0, ssem1, dsem0, dsem1, gsem0, gsem1):
        cid = lax.axis_index("c")
        sid = lax.axis_index("s")
        wid = cid * NS + sid
        ebase = wid * EPW
        r0 = pl.multiple_of(sid * RPS, 8)
        pltpu.sync_copy(zeros_hbm.at[pl.ds(r0, RPS)], acc.at[pl.ds(r0, RPS)])

        def src_start(j, sv, ssem):
            base = pl.multiple_of(ebase + j * CH, 8)
            pltpu.async_copy(src_hbm.at[pl.ds(base, CH)], sv, ssem)

        def src_wait(sv, ssem):
            pltpu.make_async_copy(src_hbm.at[pl.ds(0, CH)], sv, ssem).wait()

        def dst_start(j, dv, dsem):
            base = pl.multiple_of(ebase + j * CH, 8)
            pltpu.async_copy(dst_hbm.at[pl.ds(base, CH)], dv, dsem)

        def dst_wait(dv, dsem):
            pltpu.make_async_copy(dst_hbm.at[pl.ds(0, CH)], dv, dsem).wait()

        def gather_start(sv, rv, gsem):
            pltpu.async_copy(z_hbm.at[sv], rv, gsem)

        def gather_wait(sv, rv, gsem):
            pltpu.make_async_copy(z_hbm.at[sv], rv, gsem).wait()

        plsc.subcore_barrier()

        src_start(0, sidx0, ssem0)
        dst_start(0, didx0, dsem0)
        src_start(1, sidx1, ssem1)
        dst_start(1, didx1, dsem1)
        src_wait(sidx0, ssem0)
        gather_start(sidx0, rows0, gsem0)

        def body(i, carry):
            j0 = 2 * i
            j1 = j0 + 1
            last = i >= NCHUNK // 2 - 1
            src_wait(sidx1, ssem1)
            gather_start(sidx1, rows1, gsem1)
            gather_wait(sidx0, rows0, gsem0)

            @pl.when(jnp.logical_not(last))
            def _():
                src_start(j0 + 2, sidx0, ssem0)

            dst_wait(didx0, dsem0)
            pltpu.sync_copy(rows0, acc.at[didx0], add=True)

            @pl.when(jnp.logical_not(last))
            def _():
                dst_start(j0 + 2, didx0, dsem0)
                src_wait(sidx0, ssem0)
                gather_start(sidx0, rows0, gsem0)

            gather_wait(sidx1, rows1, gsem1)

            @pl.when(jnp.logical_not(last))
            def _():
                src_start(j1 + 2, sidx1, ssem1)

            dst_wait(didx1, dsem1)
            pltpu.sync_copy(rows1, acc.at[didx1], add=True)

            @pl.when(jnp.logical_not(last))
            def _():
                dst_start(j1 + 2, didx1, dsem1)

            return carry

        lax.fori_loop(0, NCHUNK // 2, body, 0)
        plsc.subcore_barrier()
        pltpu.sync_copy(acc.at[pl.ds(r0, RPS)], out_hbm.at[pl.ds(cid * NP + r0, RPS)])

    return _sc_agg


def _tc_prescale_body(c0_ref, c1_ref, x_ref, z_ref):
    deg = c0_ref[...] + c1_ref[...] + 1.0
    dinv = lax.rsqrt(deg)
    z_ref[0:N, :] = x_ref[...] * dinv
    z_ref[N:NP, :] = jnp.zeros((NP - N, D), jnp.float32)


def _tc_final_body(aggp_ref, x_ref, c0_ref, c1_ref, w_ref, b_ref, out_ref):
    deg = c0_ref[...] + c1_ref[...] + 1.0
    dinv = lax.rsqrt(deg)
    g = aggp_ref[0:N, :] + aggp_ref[NP : NP + N, :]
    v = g * dinv + x_ref[...] * (dinv * dinv)
    out_ref[...] = (
        jnp.dot(v, w_ref[...], preferred_element_type=jnp.float32) + b_ref[...]
    )


def kernel(x, edge_index, W_mu, b_mu, W_logstd, b_logstd):
    pad = N + (jnp.arange(EP - E, dtype=jnp.int32) % (NP - N))
    srcp = jnp.concatenate([edge_index[0], pad])
    dstp = jnp.concatenate([edge_index[1], pad])
    ones1 = jnp.ones((CH,), jnp.float32)
    zeros1 = jnp.zeros((NP,), jnp.float32)
    zerosD = jnp.zeros((NP, D), jnp.float32)

    dst3 = dstp.reshape(NW, NCHUNK, CH)
    degp = _make_sc_degree()(dst3, ones1, zeros1)
    c0 = degp[0:N].reshape(N, 1)
    c1 = degp[NP : NP + N].reshape(N, 1)
    z = pl.pallas_call(
        _tc_prescale_body,
        out_shape=jax.ShapeDtypeStruct((NP, D), jnp.float32),
    )(c0, c1, x)
    aggp = _make_sc_agg()(srcp, dstp, z, zerosD)

    W_cat = jnp.concatenate([W_mu, W_logstd], axis=1)
    b_cat = jnp.concatenate([b_mu, b_logstd]).reshape(1, 2 * D_OUT)
    out = pl.pallas_call(
        _tc_final_body,
        out_shape=jax.ShapeDtypeStruct((N, 2 * D_OUT), jnp.float32),
    )(aggp, x, c0, c1, W_cat, b_cat)
    return out[:, :D_OUT], out[:, D_OUT:]
